# baseline (speedup 1.0000x reference)
import functools

import jax
import jax.numpy as jnp
from jax import lax
from jax.experimental import pallas as pl
from jax.experimental.pallas import tpu as pltpu
from jax.experimental.pallas import tpu_sc as plsc

VOCAB = 1_000_000
EMBED = 32
BATCH = 16384
SEQ = 200
N_TOK = BATCH * SEQ

TVF = 4
TV_ROWS = VOCAB // TVF
TBLK = 2000


def _tv_body(t_ref, w_ref, b_ref, o_ref):
    o_ref[...] = (
        jnp.dot(t_ref[...], w_ref[...], preferred_element_type=jnp.float32)
        + b_ref[0]
    )


def _compute_tv(table, W, b):
    w_scaled = W.reshape(EMBED, 1) * (1.0 / SEQ)
    wd = jnp.kron(jnp.eye(TVF, dtype=jnp.float32), w_scaled)
    b_scaled = b * (1.0 / SEQ)
    tv2d = pl.pallas_call(
        _tv_body,
        grid=(TV_ROWS // TBLK,),
        in_specs=[
            pl.BlockSpec((TBLK, TVF * EMBED), lambda i: (i, 0)),
            pl.BlockSpec((TVF * EMBED, TVF), lambda i: (0, 0)),
            pl.BlockSpec(memory_space=pltpu.SMEM),
        ],
        out_specs=pl.BlockSpec((TBLK, TVF), lambda i: (i, 0)),
        out_shape=jax.ShapeDtypeStruct((TV_ROWS, TVF), jnp.float32),
    )(table.reshape(TV_ROWS, TVF * EMBED), wd, b_scaled)
    return tv2d.reshape(VOCAB)


NGRP_T = BATCH // 128
TR_GRPS = 8


def _tr_body(i_ref, o_ref):
    for g in range(TR_GRPS):
        o_ref[g * SEQ:(g + 1) * SEQ, :] = i_ref[g * 128:(g + 1) * 128, :].T


def _permute_idx(idx2d):
    out = pl.pallas_call(
        _tr_body,
        grid=(NGRP_T // TR_GRPS,),
        in_specs=[pl.BlockSpec((TR_GRPS * 128, SEQ), lambda i: (i, 0))],
        out_specs=pl.BlockSpec((TR_GRPS * SEQ, 128), lambda i: (i, 0)),
        out_shape=jax.ShapeDtypeStruct((NGRP_T * SEQ, 128), jnp.int32),
    )(idx2d)
    return out.reshape(N_TOK)


NC, NS = 2, 16
NW = NC * NS
ROWS_W = BATCH // NW
CHUNK_ROWS = 128
NCHUNK = ROWS_W // CHUNK_ROWS
CHUNK_IDX = CHUNK_ROWS * SEQ
NACC = CHUNK_ROWS // 16


def _make_pool():
    mesh = plsc.VectorSubcoreMesh(
        core_axis_name="c", subcore_axis_name="s", num_cores=NC, num_subcores=NS
    )
    @functools.partial(
        pl.kernel,
        mesh=mesh,
        out_type=jax.ShapeDtypeStruct((BATCH,), jnp.float32),
        scratch_types=[
            pltpu.VMEM((CHUNK_IDX,), jnp.int32),
            pltpu.VMEM((CHUNK_IDX,), jnp.float32),
            pltpu.VMEM((CHUNK_ROWS,), jnp.float32),
            pltpu.SemaphoreType.DMA,
        ],
    )
    def pool(idx_hbm, tv_hbm, out_hbm, idx_v, g_v, o_v, sem):
        wid = lax.axis_index("s") * NC + lax.axis_index("c")
        for c in range(NCHUNK):
            base = wid * (NCHUNK * CHUNK_IDX) + c * CHUNK_IDX
            pltpu.sync_copy(idx_hbm.at[pl.ds(base, CHUNK_IDX)], idx_v)
            pltpu.async_copy(tv_hbm.at[idx_v], g_v, sem).wait()

            for j in range(NACC):
                def tok_add(t, acc, j=j):
                    return acc + g_v[pl.ds(t * CHUNK_ROWS + j * 16, 16)]
                acc = lax.fori_loop(
                    0, SEQ, tok_add, jnp.zeros((16,), jnp.float32)
                )
                o_v[pl.ds(j * 16, 16)] = 1.0 / (1.0 + jnp.exp(-acc))

            pltpu.sync_copy(
                o_v, out_hbm.at[pl.ds(wid * ROWS_W + c * CHUNK_ROWS, CHUNK_ROWS)]
            )

    return pool


_pool_kernel = _make_pool()


def kernel(inputs, table, W, b):
    tv = _compute_tv(table, W, b)
    idx = _permute_idx(inputs.astype(jnp.int32))
    out = _pool_kernel(idx, tv)
    return out.reshape(BATCH, 1)


# speedup vs baseline: 1.1666x; 1.1666x over previous
import functools

import jax
import jax.numpy as jnp
from jax import lax
from jax.experimental import pallas as pl
from jax.experimental.pallas import tpu as pltpu
from jax.experimental.pallas import tpu_sc as plsc

VOCAB = 1_000_000
EMBED = 32
BATCH = 16384
SEQ = 200
N_TOK = BATCH * SEQ

TVF = 4
TV_ROWS = VOCAB // TVF
TBLK = 2048
TV_GRID = (TV_ROWS + TBLK - 1) // TBLK
TV_OROWS = TV_GRID * TBLK * TVF // 128


def _tv_body(t_ref, w_ref, b_ref, o_ref):
    res = (
        jnp.dot(t_ref[...], w_ref[...], preferred_element_type=jnp.float32)
        + b_ref[0]
    )
    for g in range(TBLK // 128):
        o_ref[TVF * g:TVF * (g + 1), :] = res[128 * g:128 * (g + 1), :].T


def _compute_tv(table, W, b):
    w_scaled = W.reshape(EMBED, 1) * (1.0 / SEQ)
    wd = jnp.kron(jnp.eye(TVF, dtype=jnp.float32), w_scaled)
    b_scaled = b * (1.0 / SEQ)
    tv2d = pl.pallas_call(
        _tv_body,
        grid=(TV_GRID,),
        in_specs=[
            pl.BlockSpec((TBLK, TVF * EMBED), lambda i: (i, 0)),
            pl.BlockSpec((TVF * EMBED, TVF), lambda i: (0, 0)),
            pl.BlockSpec(memory_space=pltpu.SMEM),
        ],
        out_specs=pl.BlockSpec((TBLK * TVF // 128, 128), lambda i: (i, 0)),
        out_shape=jax.ShapeDtypeStruct((TV_OROWS, 128), jnp.float32),
    )(table.reshape(TV_ROWS, TVF * EMBED), wd, b_scaled)
    return tv2d.reshape(TV_OROWS * 128)


NGRP_T = BATCH // 128
TR_GRPS = 8


def _tr_body(i_ref, o_ref):
    for g in range(TR_GRPS):
        t = i_ref[g * 128:(g + 1) * 128, :].T
        o_ref[g * SEQ:(g + 1) * SEQ, :] = (
            ((t >> 9) << 9) | ((t & 3) << 7) | ((t >> 2) & 127)
        )


def _permute_idx(idx2d):
    out = pl.pallas_call(
        _tr_body,
        grid=(NGRP_T // TR_GRPS,),
        in_specs=[pl.BlockSpec((TR_GRPS * 128, SEQ), lambda i: (i, 0))],
        out_specs=pl.BlockSpec((TR_GRPS * SEQ, 128), lambda i: (i, 0)),
        out_shape=jax.ShapeDtypeStruct((NGRP_T * SEQ, 128), jnp.int32),
    )(idx2d)
    return out.reshape(N_TOK)


NC, NS = 2, 16
NW = NC * NS
ROWS_W = BATCH // NW
CHUNK_ROWS = 128
NCHUNK = ROWS_W // CHUNK_ROWS
CHUNK_IDX = CHUNK_ROWS * SEQ
NACC = CHUNK_ROWS // 16


def _make_pool():
    mesh = plsc.VectorSubcoreMesh(
        core_axis_name="c", subcore_axis_name="s", num_cores=NC, num_subcores=NS
    )
    @functools.partial(
        pl.kernel,
        mesh=mesh,
        out_type=jax.ShapeDtypeStruct((BATCH,), jnp.float32),
        scratch_types=[
            pltpu.VMEM((CHUNK_IDX,), jnp.int32),
            pltpu.VMEM((CHUNK_IDX,), jnp.float32),
            pltpu.VMEM((CHUNK_ROWS,), jnp.float32),
            pltpu.SemaphoreType.DMA,
        ],
    )
    def pool(idx_hbm, tv_hbm, out_hbm, idx_v, g_v, o_v, sem):
        wid = lax.axis_index("s") * NC + lax.axis_index("c")
        for c in range(NCHUNK):
            base = wid * (NCHUNK * CHUNK_IDX) + c * CHUNK_IDX
            pltpu.sync_copy(idx_hbm.at[pl.ds(base, CHUNK_IDX)], idx_v)
            pltpu.async_copy(tv_hbm.at[idx_v], g_v, sem).wait()

            for j in range(NACC):
                def tok_add(t, acc, j=j):
                    return acc + g_v[pl.ds(t * CHUNK_ROWS + j * 16, 16)]
                acc = lax.fori_loop(
                    0, SEQ, tok_add, jnp.zeros((16,), jnp.float32)
                )
                o_v[pl.ds(j * 16, 16)] = 1.0 / (1.0 + jnp.exp(-acc))

            pltpu.sync_copy(
                o_v, out_hbm.at[pl.ds(wid * ROWS_W + c * CHUNK_ROWS, CHUNK_ROWS)]
            )

    return pool


_pool_kernel = _make_pool()


def kernel(inputs, table, W, b):
    tv = _compute_tv(table, W, b)
    idx = _permute_idx(inputs.astype(jnp.int32))
    out = _pool_kernel(idx, tv)
    return out.reshape(BATCH, 1)
